# Initial kernel scaffold; baseline (speedup 1.0000x reference)
import jax
import jax.numpy as jnp
from jax.experimental import pallas as pl


def kernel(input, tables):
    raise NotImplementedError("write your pallas kernel here")


# speedup vs baseline: 1.2135x; 1.2135x over previous
import functools

import jax
import jax.numpy as jnp
from jax import lax
from jax.experimental import pallas as pl
from jax.experimental.pallas import tpu as pltpu
from jax.experimental.pallas import tpu_sc as plsc

_GW = 128
_DEPTH = 3


@functools.lru_cache(maxsize=None)
def _make_gather(N, D, NW):
    per_w = N // NW
    G = 8
    C = G * _GW
    n_chunks = per_w // C
    n_blocks = per_w // _GW
    mesh = plsc.VectorSubcoreMesh(core_axis_name="c", subcore_axis_name="s")

    @functools.partial(
        pl.kernel,
        mesh=mesh,
        out_type=jax.ShapeDtypeStruct((N, D), jnp.float32),
        compiler_params=pltpu.CompilerParams(use_tc_tiling_on_sc=False),
        scratch_types=[
            pltpu.VMEM((n_blocks, _GW), jnp.int32),
            pltpu.VMEM((_DEPTH, C, D), jnp.float32),
            [pltpu.SemaphoreType.DMA] * _DEPTH,
            [pltpu.SemaphoreType.DMA] * _DEPTH,
        ],
    )
    def k(idx_hbm, tab_hbm, out_hbm, idx_v, rows_v, gsem, osem):
        wid = lax.axis_index("s") * 2 + lax.axis_index("c")
        base = wid * per_w
        rbase = wid * n_blocks

        pltpu.sync_copy(idx_hbm.at[pl.ds(rbase, n_blocks)], idx_v)

        def fire_gathers(c):
            b = c % _DEPTH
            return [
                pltpu.async_copy(
                    tab_hbm.at[idx_v.at[c * G + g]],
                    rows_v.at[b].at[pl.ds(g * _GW, _GW)],
                    gsem[b],
                )
                for g in range(G)
            ]

        def fire_owrite(c):
            b = c % _DEPTH
            return pltpu.async_copy(
                rows_v.at[b], out_hbm.at[pl.ds(base + c * C, C)], osem[b]
            )

        gd, od = {}, {}
        for c in range(min(_DEPTH - 1, n_chunks)):
            gd[c] = fire_gathers(c)
        for c in range(n_chunks):
            nxt = c + _DEPTH - 1
            if nxt < n_chunks:
                if nxt - _DEPTH >= 0:
                    od.pop(nxt - _DEPTH).wait()
                gd[nxt] = fire_gathers(nxt)
            for d in gd.pop(c):
                d.wait()
            od[c] = fire_owrite(c)
        for c in sorted(od):
            od.pop(c).wait()

    return k


def kernel(input, tables):
    B, F = input.shape
    _, V, D = tables.shape
    N = B * F
    info = plsc.get_sparse_core_info()
    NW = info.num_cores * info.num_subcores
    flat_idx = (input + jnp.arange(F, dtype=input.dtype) * V).reshape(N // _GW, _GW)
    flat_tab = tables.reshape(F * V, D)
    out = _make_gather(N, D, NW)(flat_idx, flat_tab)
    return out.reshape(B, F * D)
